# Initial kernel scaffold; baseline (speedup 1.0000x reference)
import jax
import jax.numpy as jnp
from jax.experimental import pallas as pl


def kernel(mol_a_reprs, node_feats, edge_feats, edge_index, params):
    raise NotImplementedError("write your pallas kernel here")


# speedup vs baseline: 4.1244x; 4.1244x over previous
import jax
import jax.numpy as jnp
import numpy as np
from jax.experimental import pallas as pl

_B = 2048
_NP = 40
_EP = 120
_NF = 64
_EF = 16
_H = 128
_EH = 64
_ATT = 128
_MAX = 50
_STEPS = 3
_G = 8

_INV_D = 1.0 / np.sqrt(float(_ATT))


def _dot(a, b):
    return jax.lax.dot_general(a, b, (((1,), (0,)), ((), ())),
                               preferred_element_type=jnp.float32)


def _dotT0(a, b):
    return jax.lax.dot_general(a, b, (((0,), (0,)), ((), ())),
                               preferred_element_type=jnp.float32)


def _dotT1(a, b):
    return jax.lax.dot_general(a, b, (((1,), (1,)), ((), ())),
                               preferred_element_type=jnp.float32)


def _chunk_body(mol_ref, nodes_ref, edges_ref, src_ref, dst_ref,
                w_ni, b_ni, w_ei, b_ei, wm_n, wm_e, b_msg, wu_h, wu_a, b_upd,
                wk1, bk1, wk2, bk2, wq1, bq1, wq2, bq2, wv1, bv1, wv2, bv2,
                uk1, ubk1, uk2, ubk2, uq1, ubq1, uq2, ubq2,
                wc1t, wc1pad, bc1, wc2, bc2,
                out_ref):
    G = _G
    NC = G * _NP
    f32 = jnp.float32
    relu = lambda x: jnp.maximum(x, 0.0)

    def mlp(x, w1, b1, w2, b2):
        return _dot(relu(_dot(x, w1[...]) + b1[...]), w2[...]) + b2[...]

    chunk_base = pl.program_id(0) * NC
    src = (src_ref[...] - chunk_base)[:, :, None]
    dst = (dst_ref[...] - chunk_base)[:, :, None]
    niota = jax.lax.broadcasted_iota(jnp.int32, (G, _EP, NC), 2)
    oh_src = (niota == src).astype(f32).reshape(G * _EP, NC)
    oh_dst = (niota == dst).astype(f32).reshape(G * _EP, NC)

    nh = relu(_dot(nodes_ref[...], w_ni[...]) + b_ni[...])
    ehid = relu(_dot(edges_ref[...], w_ei[...]) + b_ei[...])
    ehc = _dot(ehid, wm_e[...])
    for _ in range(_STEPS):
        gath = _dot(oh_src, nh)
        msg = relu(_dot(gath, wm_n[...]) + ehc + b_msg[...])
        agg = _dotT0(oh_dst, msg)
        nh = relu(_dot(nh, wu_h[...]) + _dot(agg, wu_a[...]) + b_upd[...])

    Km = mlp(nh, wk1, bk1, wk2, bk2)
    Qm = mlp(nh, wq1, bq1, wq2, bq2)
    Vm = mlp(nh, wv1, bv1, wv2, bv2)
    kqt = _dotT1(Km, Qm)
    same_g = (jax.lax.broadcasted_iota(jnp.int32, (NC, NC), 0) // _NP ==
              jax.lax.broadcasted_iota(jnp.int32, (NC, NC), 1) // _NP)
    s = jnp.where(same_g, kqt, -1e9) * _INV_D
    m = jnp.max(s, axis=1, keepdims=True)
    p = jnp.exp(s - m)
    attn = p / jnp.sum(p, axis=1, keepdims=True)
    corr = _dot(attn, Vm)

    v_pad = _dot(relu(bv1[...]), wv2[...]) + bv2[...]
    ones_bd = (jax.lax.broadcasted_iota(jnp.int32, (G, NC), 1) // _NP ==
               jax.lax.broadcasted_iota(jnp.int32, (G, NC), 0)).astype(f32)
    vsum = _dot(ones_bd, Vm)
    corr_pad = (vsum + float(_MAX - _NP) * v_pad) * (1.0 / _MAX)

    K2 = mlp(corr, uk1, ubk1, uk2, ubk2)
    K2p = mlp(corr_pad, uk1, ubk1, uk2, ubk2)
    Q2 = mlp(mol_ref[...], uq1, ubq1, uq2, ubq2)
    Q2e = _dotT0(ones_bd, Q2)
    logit_col = jnp.sum(K2 * Q2e, axis=1, keepdims=True) * _INV_D
    pad_logit = jnp.sum(K2p * Q2, axis=1, keepdims=True) * _INV_D

    h = relu(_dot(ones_bd, logit_col * wc1t[...])
             + pad_logit * wc1pad[...] + bc1[...])
    out_ref[...] = jax.nn.sigmoid(_dot(h, wc2[...]) + bc2[...])


def kernel(mol_a_reprs, node_feats, edge_feats, edge_index, params):
    p = params
    f32 = jnp.float32
    row = lambda b: b.reshape(1, -1).astype(f32)

    src2d = edge_index[0].reshape(_B, _EP).astype(jnp.int32)
    dst2d = edge_index[1].reshape(_B, _EP).astype(jnp.int32)

    wq1, bq1, wq2, bq2 = p["Wq"]
    wk1, bk1, wk2, bk2 = p["Wk"]
    wv1, bv1, wv2, bv2 = p["Wv"]
    uk1, ubk1, uk2, ubk2 = p["Uk"]
    uq1, ubq1, uq2, ubq2 = p["Uq"]
    c1, cb1, c2, cb2 = p["Cs"]

    weights = (
        p["W_ni"], row(p["b_ni"]), p["W_ei"], row(p["b_ei"]),
        p["W_msg"][:_H], p["W_msg"][_H:], row(p["b_msg"]),
        p["W_upd"][:_H], p["W_upd"][_H:], row(p["b_upd"]),
        wk1, row(bk1), wk2, row(bk2),
        wq1, row(bq1), wq2, row(bq2),
        wv1, row(bv1), wv2, row(bv2),
        uk1, row(ubk1), uk2, row(ubk2),
        uq1, row(ubq1), uq2, row(ubq2),
        jnp.tile(c1[:_NP], (_G, 1)),
        jnp.sum(c1[_NP:], axis=0, keepdims=True),
        row(cb1), c2[:, :_NP], row(cb2[:_NP]),
    )

    grid = (_B // _G,)
    data_specs = [
        pl.BlockSpec((_G, 256), lambda i: (i, 0)),
        pl.BlockSpec((_G * _NP, _NF), lambda i: (i, 0)),
        pl.BlockSpec((_G * _EP, _EF), lambda i: (i, 0)),
        pl.BlockSpec((_G, _EP), lambda i: (i, 0)),
        pl.BlockSpec((_G, _EP), lambda i: (i, 0)),
    ]
    w_specs = [pl.BlockSpec(w.shape, lambda i: (0,) * w.ndim) for w in weights]

    out2d = pl.pallas_call(
        _chunk_body,
        grid=grid,
        in_specs=data_specs + w_specs,
        out_specs=pl.BlockSpec((_G, _NP), lambda i: (i, 0)),
        out_shape=jax.ShapeDtypeStruct((_B, _NP), f32),
    )(mol_a_reprs, node_feats, edge_feats, src2d, dst2d, *weights)
    return out2d.reshape(-1)
